# Initial kernel scaffold; baseline (speedup 1.0000x reference)
import jax
import jax.numpy as jnp
from jax.experimental import pallas as pl


def kernel(x, token_table, pos_table):
    raise NotImplementedError("write your pallas kernel here")


# speedup vs baseline: 3.6266x; 3.6266x over previous
import functools

import jax
import jax.numpy as jnp
from jax import lax
from jax.experimental import pallas as pl
from jax.experimental.pallas import tpu as pltpu
from jax.experimental.pallas import tpu_sc as plsc

BATCH = 4096
MAXLEN = 200
VOCAB = 100000
D = 32
TOT = BATCH * MAXLEN

NC, NS, L = 2, 16, 16
NW = NC * NS
RPW = TOT // NW
CH = 1024
NCH = RPW // CH
GS = 128
NG = CH // GS
PTILE = 7

_mesh = plsc.VectorSubcoreMesh(
    core_axis_name="c", subcore_axis_name="s", num_cores=NC, num_subcores=NS
)


@functools.partial(
    pl.kernel,
    out_type=jax.ShapeDtypeStruct((TOT, D), jnp.float32),
    mesh=_mesh,
    compiler_params=pltpu.CompilerParams(use_tc_tiling_on_sc=False),
    scratch_types=[
        pltpu.VMEM((NG, GS), jnp.int32),
        pltpu.VMEM((CH, D), jnp.float32),
        pltpu.VMEM((PTILE * MAXLEN, D), jnp.float32),
        pltpu.SemaphoreType.DMA,
    ],
)
def _embed(x_hbm, tok_hbm, pos_hbm, out_hbm, idx_v, rows_v, pos_v, sem):
    wid = lax.axis_index("s") * NC + lax.axis_index("c")

    for t in range(PTILE):
        pltpu.sync_copy(pos_hbm, pos_v.at[pl.ds(t * MAXLEN, MAXLEN)])

    base_w = wid * RPW

    @pl.loop(0, NCH)
    def _chunk(k):
        base = base_w + k * CH
        r0 = wid * (RPW // GS) + k * NG
        pltpu.sync_copy(x_hbm.at[pl.ds(r0, NG)], idx_v)

        copies = []
        for j in range(NG):
            copies.append(
                pltpu.async_copy(
                    tok_hbm.at[idx_v.at[j]],
                    rows_v.at[pl.ds(j * GS, GS)],
                    sem,
                )
            )
        for c in copies:
            c.wait()

        ph = lax.rem(k * CH, MAXLEN)

        @pl.loop(0, CH, unroll=4)
        def _row(j):
            pr = ph + j
            for h in range(2):
                v = pos_v[pr, pl.ds(h * L, L)]
                plsc.addupdate(rows_v.at[j, pl.ds(h * L, L)], v)

        pltpu.sync_copy(rows_v, out_hbm.at[pl.ds(base, CH)])


def kernel(x, token_table, pos_table):
    x2 = x.reshape(TOT // GS, GS).astype(jnp.int32)
    out = _embed(x2, token_table, pos_table)
    return out.reshape(BATCH, MAXLEN, D)


# speedup vs baseline: 4.2033x; 1.1590x over previous
import functools

import jax
import jax.numpy as jnp
from jax import lax
from jax.experimental import pallas as pl
from jax.experimental.pallas import tpu as pltpu
from jax.experimental.pallas import tpu_sc as plsc

BATCH = 4096
MAXLEN = 200
VOCAB = 100000
D = 32
TOT = BATCH * MAXLEN

NC, NS, L = 2, 16, 16
NW = NC * NS
RPW = TOT // NW
CH = 256
NCH = RPW // CH
GS = 128
NGc = CH // GS
NBUF = 5
FA = 3
PTILE = 3

_mesh = plsc.VectorSubcoreMesh(
    core_axis_name="c", subcore_axis_name="s", num_cores=NC, num_subcores=NS
)


@functools.partial(
    pl.kernel,
    out_type=jax.ShapeDtypeStruct((TOT, D), jnp.float32),
    mesh=_mesh,
    compiler_params=pltpu.CompilerParams(use_tc_tiling_on_sc=False),
    scratch_types=[
        pltpu.VMEM((RPW // GS, GS), jnp.int32),
        [pltpu.VMEM((CH, D), jnp.float32)] * NBUF,
        pltpu.VMEM((PTILE * MAXLEN, D), jnp.float32),
        [pltpu.SemaphoreType.DMA] * NBUF,
        [pltpu.SemaphoreType.DMA] * NBUF,
    ],
)
def _embed(x_hbm, tok_hbm, pos_hbm, out_hbm, idx_all, rows, pos_v, semg, sems):
    wid = lax.axis_index("s") * NC + lax.axis_index("c")
    base_w = wid * RPW

    for t in range(PTILE):
        pltpu.sync_copy(pos_hbm, pos_v.at[pl.ds(t * MAXLEN, MAXLEN)])
    pltpu.sync_copy(x_hbm.at[pl.ds(wid * (RPW // GS), RPW // GS)], idx_all)

    def fire(k, b):
        for j in range(NGc):
            pltpu.async_copy(
                tok_hbm.at[idx_all.at[k * NGc + j]],
                rows[b].at[pl.ds(j * GS, GS)],
                semg[b],
            )

    def drain_store(b):
        pltpu.make_async_copy(rows[b], out_hbm.at[pl.ds(0, CH)], sems[b]).wait()

    def proc(k, b):
        pltpu.make_async_copy(out_hbm.at[pl.ds(0, CH)], rows[b], semg[b]).wait()
        ph = lax.rem(k * CH, MAXLEN)

        @pl.loop(0, CH, unroll=8)
        def _row(j):
            pr = ph + j
            for h in range(2):
                plsc.addupdate(rows[b].at[j, pl.ds(h * L, L)], pos_v[pr, pl.ds(h * L, L)])

        pltpu.async_copy(rows[b], out_hbm.at[pl.ds(base_w + k * CH, CH)], sems[b])

    for i in range(FA):
        fire(i, i)

    @pl.loop(0, NCH // NBUF)
    def _grp(p):
        k0 = p * NBUF
        for i in range(NBUF):
            k = k0 + i
            proc(k, i)
            kf = k + FA
            bf = (i + FA) % NBUF

            @pl.when(jnp.logical_and(kf >= NBUF, kf < NCH))
            def _drain():
                drain_store(bf)

            @pl.when(kf < NCH)
            def _fire():
                fire(kf, bf)

    for i in range(NBUF):
        drain_store(i)


def kernel(x, token_table, pos_table):
    x2 = x.reshape(TOT // GS, GS).astype(jnp.int32)
    out = _embed(x2, token_table, pos_table)
    return out.reshape(BATCH, MAXLEN, D)


# speedup vs baseline: 5.1917x; 1.2352x over previous
import functools

import jax
import jax.numpy as jnp
from jax import lax
from jax.experimental import pallas as pl
from jax.experimental.pallas import tpu as pltpu
from jax.experimental.pallas import tpu_sc as plsc

BATCH = 4096
MAXLEN = 200
VOCAB = 100000
D = 32
TOT = BATCH * MAXLEN

NC, NS, L = 2, 16, 16
NW = NC * NS
RPW = TOT // NW
CH = MAXLEN
NCH = RPW // CH
GS = 100
NGc = CH // GS
NBUF = 8
FA = 3

_mesh = plsc.VectorSubcoreMesh(
    core_axis_name="c", subcore_axis_name="s", num_cores=NC, num_subcores=NS
)


@functools.partial(
    pl.kernel,
    out_type=jax.ShapeDtypeStruct((TOT, D), jnp.float32),
    mesh=_mesh,
    compiler_params=pltpu.CompilerParams(use_tc_tiling_on_sc=False),
    scratch_types=[
        pltpu.VMEM((RPW // GS, GS), jnp.int32),
        [pltpu.VMEM((CH, D), jnp.float32)] * NBUF,
        pltpu.VMEM_SHARED((MAXLEN, D), jnp.float32),
        [pltpu.SemaphoreType.DMA] * NBUF,
        [pltpu.SemaphoreType.DMA] * NBUF,
    ],
)
def _embed(x_hbm, tok_hbm, pos_hbm, out_hbm, idx_all, rows, pos_v, semg, sems):
    wid = lax.axis_index("s") * NC + lax.axis_index("c")
    base_w = wid * RPW

    @pl.when(lax.axis_index("s") == 0)
    def _stage_pos():
        pltpu.sync_copy(pos_hbm, pos_v)

    pltpu.sync_copy(x_hbm.at[pl.ds(wid * (RPW // GS), RPW // GS)], idx_all)
    plsc.subcore_barrier()

    def fire(k, b):
        pltpu.sync_copy(pos_v, rows[b])
        for j in range(NGc):
            pltpu.async_copy(
                tok_hbm.at[idx_all.at[k * NGc + j]],
                rows[b].at[pl.ds(j * GS, GS)],
                semg[b],
                add=True,
            )

    def drain_store(b):
        pltpu.make_async_copy(rows[b], out_hbm.at[pl.ds(0, CH)], sems[b]).wait()

    def proc(k, b):
        pltpu.make_async_copy(out_hbm.at[pl.ds(0, CH)], rows[b], semg[b]).wait()
        pltpu.async_copy(rows[b], out_hbm.at[pl.ds(base_w + k * CH, CH)], sems[b])

    for i in range(FA):
        fire(i, i)

    @pl.loop(0, NCH // NBUF)
    def _grp(p):
        k0 = p * NBUF
        for i in range(NBUF):
            k = k0 + i
            proc(k, i)
            kf = k + FA
            bf = (i + FA) % NBUF

            @pl.when(jnp.logical_and(kf >= NBUF, kf < NCH))
            def _drain():
                drain_store(bf)

            @pl.when(kf < NCH)
            def _fire():
                fire(kf, bf)

    for i in range(NBUF):
        drain_store(i)


def kernel(x, token_table, pos_table):
    x2 = x.reshape(TOT // GS, GS).astype(jnp.int32)
    out = _embed(x2, token_table, pos_table)
    return out.reshape(BATCH, MAXLEN, D)
